# baseline (speedup 1.0000x reference)
import functools

import jax
import jax.numpy as jnp
from jax import lax
from jax.experimental import pallas as pl
from jax.experimental.pallas import tpu as pltpu
from jax.experimental.pallas import tpu_sc as plsc

_D = 128
_N = 4096 * 26
_NC, _NS = 2, 16
_NW = _NC * _NS
_BPW = _N // _NW
_C = 416
_NCHUNK = _BPW // _C
_NB = 2

_mesh = plsc.VectorSubcoreMesh(core_axis_name="c", subcore_axis_name="s")


@functools.partial(
    pl.kernel,
    out_type=jax.ShapeDtypeStruct((_N, _D), jnp.float32),
    mesh=_mesh,
    scratch_types=[
        pltpu.VMEM((_BPW,), jnp.int32),
        pltpu.VMEM((_NB, _C, _D), jnp.float32),
        pltpu.SemaphoreType.DMA,
        pltpu.SemaphoreType.DMA,
    ],
)
def _gather_kernel(idx_hbm, table_hbm, out_hbm, idx_v, rows_v, sem_g, sem_w):
    wid = lax.axis_index("s") * _NC + lax.axis_index("c")
    base = wid * _BPW
    pltpu.sync_copy(idx_hbm.at[pl.ds(base, _BPW)], idx_v)

    def gather(j):
        return pltpu.async_copy(
            table_hbm.at[idx_v.at[pl.ds(j * _C, _C)]], rows_v.at[j % _NB], sem_g)

    def writeback(j):
        return pltpu.async_copy(
            rows_v.at[j % _NB], out_hbm.at[pl.ds(base + j * _C, _C)], sem_w)

    g = [None] * _NCHUNK
    w = [None] * _NCHUNK
    g[0] = gather(0)
    for j in range(_NCHUNK):
        if j + 1 < _NCHUNK:
            if j + 1 >= _NB:
                w[j + 1 - _NB].wait()
            g[j + 1] = gather(j + 1)
        g[j].wait()
        w[j] = writeback(j)
    for j in range(max(0, _NCHUNK - _NB), _NCHUNK):
        w[j].wait()


def kernel(x, embed_matrix):
    idx = x.reshape(-1).astype(jnp.int32)
    out = _gather_kernel(idx, embed_matrix)
    return out.reshape(x.shape[0], x.shape[1], _D)


# speedup vs baseline: 1.2866x; 1.2866x over previous
import functools

import jax
import jax.numpy as jnp
from jax import lax
from jax.experimental import pallas as pl
from jax.experimental.pallas import tpu as pltpu
from jax.experimental.pallas import tpu_sc as plsc

_D = 128
_N = 4096 * 26
_NC, _NS = 2, 16
_NW = _NC * _NS
_BPW = _N // _NW
_C = 416
_NCHUNK = _BPW // _C
_NB = 2

_V = 1000

_mesh = plsc.VectorSubcoreMesh(core_axis_name="c", subcore_axis_name="s")


@functools.partial(
    pl.kernel,
    out_type=jax.ShapeDtypeStruct((_N, _D), jnp.float32),
    mesh=_mesh,
    scratch_types=[
        pltpu.VMEM((_BPW,), jnp.int32),
        pltpu.VMEM((_NB, _C, _D), jnp.float32),
        pltpu.VMEM_SHARED((_V, _D), jnp.float32),
        pltpu.SemaphoreType.DMA,
        pltpu.SemaphoreType.DMA,
    ],
)
def _gather_kernel(idx_hbm, table_hbm, out_hbm, idx_v, rows_v, table_sh,
                   sem_g, sem_w):
    sid = lax.axis_index("s")
    wid = sid * _NC + lax.axis_index("c")
    base = wid * _BPW

    @pl.when(sid == 0)
    def _():
        pltpu.sync_copy(table_hbm, table_sh)

    pltpu.sync_copy(idx_hbm.at[pl.ds(base, _BPW)], idx_v)
    plsc.subcore_barrier()

    def gather(j):
        return pltpu.async_copy(
            table_sh.at[idx_v.at[pl.ds(j * _C, _C)]], rows_v.at[j % _NB], sem_g)

    def writeback(j):
        return pltpu.async_copy(
            rows_v.at[j % _NB], out_hbm.at[pl.ds(base + j * _C, _C)], sem_w)

    g = [None] * _NCHUNK
    w = [None] * _NCHUNK
    g[0] = gather(0)
    for j in range(_NCHUNK):
        if j + 1 < _NCHUNK:
            if j + 1 >= _NB:
                w[j + 1 - _NB].wait()
            g[j + 1] = gather(j + 1)
        g[j].wait()
        w[j] = writeback(j)
    for j in range(max(0, _NCHUNK - _NB), _NCHUNK):
        w[j].wait()


def kernel(x, embed_matrix):
    idx = x.reshape(-1).astype(jnp.int32)
    out = _gather_kernel(idx, embed_matrix)
    return out.reshape(x.shape[0], x.shape[1], _D)


# speedup vs baseline: 1.9275x; 1.4981x over previous
import functools

import jax
import jax.numpy as jnp
from jax import lax
from jax.experimental import pallas as pl
from jax.experimental.pallas import tpu as pltpu
from jax.experimental.pallas import tpu_sc as plsc

_D = 128
_B = 4096
_F = 26
_FP = 32
_NP = _B * _FP
_V = 1000
_NC, _NS = 2, 16
_NW = _NC * _NS
_BPW = _NP // _NW
_C = 256
_NCHUNK = _BPW // _C
_NB = 2

_mesh = plsc.VectorSubcoreMesh(core_axis_name="c", subcore_axis_name="s")


@functools.partial(
    pl.kernel,
    out_type=jax.ShapeDtypeStruct((_NP, _D), jnp.float32),
    mesh=_mesh,
    scratch_types=[
        pltpu.VMEM((_BPW,), jnp.int32),
        pltpu.VMEM((_NB, _C, _D), jnp.float32),
        pltpu.VMEM_SHARED((_V, _D), jnp.float32),
        pltpu.SemaphoreType.DMA,
        pltpu.SemaphoreType.DMA,
    ],
)
def _gather_kernel(idx_hbm, table_hbm, out_hbm, idx_v, rows_v, table_sh,
                   sem_g, sem_w):
    sid = lax.axis_index("s")
    wid = sid * _NC + lax.axis_index("c")
    base = wid * _BPW

    @pl.when(sid == 0)
    def _():
        pltpu.sync_copy(table_hbm, table_sh)

    pltpu.sync_copy(idx_hbm.at[pl.ds(base, _BPW)], idx_v)
    plsc.subcore_barrier()

    def gather(j):
        return pltpu.async_copy(
            table_sh.at[idx_v.at[pl.ds(j * _C, _C)]], rows_v.at[j % _NB], sem_g)

    def writeback(j):
        return pltpu.async_copy(
            rows_v.at[j % _NB], out_hbm.at[pl.ds(base + j * _C, _C)], sem_w)

    g = [None] * _NCHUNK
    w = [None] * _NCHUNK
    g[0] = gather(0)
    for j in range(_NCHUNK):
        if j + 1 < _NCHUNK:
            if j + 1 >= _NB:
                w[j + 1 - _NB].wait()
            g[j + 1] = gather(j + 1)
        g[j].wait()
        w[j] = writeback(j)
    for j in range(max(0, _NCHUNK - _NB), _NCHUNK):
        w[j].wait()


def kernel(x, embed_matrix):
    xp = jnp.pad(x.astype(jnp.int32), ((0, 0), (0, _FP - _F))).reshape(-1)
    out = _gather_kernel(xp, embed_matrix)
    return out.reshape(_B, _FP, _D)[:, :_F, :]


# speedup vs baseline: 1.9380x; 1.0055x over previous
import functools

import jax
import jax.numpy as jnp
from jax import lax
from jax.experimental import pallas as pl
from jax.experimental.pallas import tpu as pltpu
from jax.experimental.pallas import tpu_sc as plsc

_D = 128
_B = 4096
_F = 26
_FP = 32
_NP = _B * _FP
_V = 1000
_NC, _NS = 2, 16
_NW = _NC * _NS
_BPW = _B // _NW
_NBATCH = 8
_C = _NBATCH * _FP
_NCHUNK = _BPW // _NBATCH
_NB = 2

_mesh = plsc.VectorSubcoreMesh(core_axis_name="c", subcore_axis_name="s")


@functools.partial(
    pl.kernel,
    out_type=jax.ShapeDtypeStruct((_B, _F, _D), jnp.float32),
    mesh=_mesh,
    compiler_params=pltpu.CompilerParams(use_tc_tiling_on_sc=True),
    scratch_types=[
        pltpu.VMEM((_BPW * _FP,), jnp.int32),
        pltpu.VMEM((_NB, _C, _D), jnp.float32),
        pltpu.VMEM_SHARED((_V, _D), jnp.float32),
        pltpu.SemaphoreType.DMA,
        pltpu.SemaphoreType.DMA,
    ],
)
def _gather_kernel(idx_hbm, table_hbm, out_hbm, idx_v, rows_v, table_sh,
                   sem_g, sem_w):
    sid = lax.axis_index("s")
    wid = sid * _NC + lax.axis_index("c")
    base_b = wid * _BPW

    @pl.when(sid == 0)
    def _():
        pltpu.sync_copy(table_hbm, table_sh)

    pltpu.sync_copy(idx_hbm.at[pl.ds(base_b * _FP, _BPW * _FP)], idx_v)
    plsc.subcore_barrier()

    def gather(j, buf):
        return pltpu.async_copy(
            table_sh.at[idx_v.at[pl.ds(j * _C, _C)]], rows_v.at[buf], sem_g)

    def writeback(j, buf):
        src = rows_v.at[buf].reshape(_NBATCH, _FP, _D).at[:, pl.ds(0, _F)]
        return pltpu.async_copy(
            src, out_hbm.at[pl.ds(base_b + j * _NBATCH, _NBATCH)], sem_w)

    def body(i, carry):
        c0 = _NB * i
        g0 = gather(c0, 0)
        g1 = gather(c0 + 1, 1)
        g0.wait()
        w0 = writeback(c0, 0)
        g1.wait()
        w1 = writeback(c0 + 1, 1)
        w0.wait()
        w1.wait()
        return carry

    lax.fori_loop(0, _NCHUNK // _NB, body, 0)


def kernel(x, embed_matrix):
    xp = jnp.pad(x.astype(jnp.int32), ((0, 0), (0, _FP - _F))).reshape(-1)
    return _gather_kernel(xp, embed_matrix)


# speedup vs baseline: 4.5781x; 2.3623x over previous
import functools

import jax
import jax.numpy as jnp
from jax import lax
from jax.experimental import pallas as pl
from jax.experimental.pallas import tpu as pltpu
from jax.experimental.pallas import tpu_sc as plsc

_D = 128
_B = 4096
_F = 26
_N = _B * _F
_V = 1000
_NC, _NS = 2, 16
_NW = _NC * _NS
_BPW = _N // _NW
_C = 416
_NCHUNK = _BPW // _C
_NB = 2

_mesh = plsc.VectorSubcoreMesh(core_axis_name="c", subcore_axis_name="s")


@functools.partial(
    pl.kernel,
    out_type=jax.ShapeDtypeStruct((_N, _D), jnp.float32),
    mesh=_mesh,
    compiler_params=pltpu.CompilerParams(use_tc_tiling_on_sc=True),
    scratch_types=[
        pltpu.VMEM((_BPW,), jnp.int32),
        pltpu.VMEM((_NB, _C, _D), jnp.float32),
        pltpu.VMEM_SHARED((_V, _D), jnp.float32),
        pltpu.SemaphoreType.DMA,
        pltpu.SemaphoreType.DMA,
    ],
)
def _gather_kernel(idx_hbm, table_hbm, out_hbm, idx_v, rows_v, table_sh,
                   sem_g, sem_w):
    sid = lax.axis_index("s")
    wid = sid * _NC + lax.axis_index("c")
    base = wid * _BPW

    @pl.when(sid == 0)
    def _():
        pltpu.sync_copy(table_hbm, table_sh)

    pltpu.sync_copy(idx_hbm.at[pl.ds(base, _BPW)], idx_v)
    plsc.subcore_barrier()

    def gather(j):
        return pltpu.async_copy(
            table_sh.at[idx_v.at[pl.ds(j * _C, _C)]], rows_v.at[j % _NB], sem_g)

    def writeback(j):
        return pltpu.async_copy(
            rows_v.at[j % _NB], out_hbm.at[pl.ds(base + j * _C, _C)], sem_w)

    g = [None] * _NCHUNK
    w = [None] * _NCHUNK
    g[0] = gather(0)
    for j in range(_NCHUNK):
        if j + 1 < _NCHUNK:
            if j + 1 >= _NB:
                w[j + 1 - _NB].wait()
            g[j + 1] = gather(j + 1)
        g[j].wait()
        w[j] = writeback(j)
    for j in range(max(0, _NCHUNK - _NB), _NCHUNK):
        w[j].wait()


def kernel(x, embed_matrix):
    idx = x.astype(jnp.int32).T.reshape(-1)
    out = _gather_kernel(idx, embed_matrix)
    return out.reshape(_F, _B, _D).transpose(1, 0, 2)


# speedup vs baseline: 4.6306x; 1.0115x over previous
import functools

import jax
import jax.numpy as jnp
from jax import lax
from jax.experimental import pallas as pl
from jax.experimental.pallas import tpu as pltpu
from jax.experimental.pallas import tpu_sc as plsc

_D = 128
_B = 4096
_F = 26
_N = _B * _F
_V = 1000
_NC, _NS = 2, 16
_NW = _NC * _NS
_BPW = _N // _NW
_C = 256
_NCHUNK = _BPW // _C
_NB = 3

_mesh = plsc.VectorSubcoreMesh(core_axis_name="c", subcore_axis_name="s")


@functools.partial(
    pl.kernel,
    out_type=jax.ShapeDtypeStruct((_N, _D), jnp.float32),
    mesh=_mesh,
    compiler_params=pltpu.CompilerParams(use_tc_tiling_on_sc=True),
    scratch_types=[
        pltpu.VMEM((_BPW,), jnp.int32),
        pltpu.VMEM((_NB, _C, _D), jnp.float32),
        pltpu.VMEM_SHARED((_V, _D), jnp.float32),
        pltpu.SemaphoreType.DMA,
        pltpu.SemaphoreType.DMA,
    ],
)
def _gather_kernel(idx_hbm, table_hbm, out_hbm, idx_v, rows_v, table_sh,
                   sem_g, sem_w):
    sid = lax.axis_index("s")
    wid = sid * _NC + lax.axis_index("c")
    base = wid * _BPW

    @pl.when(sid == 0)
    def _():
        pltpu.sync_copy(table_hbm, table_sh)

    pltpu.sync_copy(idx_hbm.at[pl.ds(base, _BPW)], idx_v)
    plsc.subcore_barrier()

    def gather(j):
        return pltpu.async_copy(
            table_sh.at[idx_v.at[pl.ds(j * _C, _C)]], rows_v.at[j % _NB], sem_g)

    def writeback(j):
        return pltpu.async_copy(
            rows_v.at[j % _NB], out_hbm.at[pl.ds(base + j * _C, _C)], sem_w)

    g = [None] * _NCHUNK
    w = [None] * _NCHUNK
    g[0] = gather(0)
    for j in range(_NCHUNK):
        if j + 1 < _NCHUNK:
            if j + 1 >= _NB:
                w[j + 1 - _NB].wait()
            g[j + 1] = gather(j + 1)
        g[j].wait()
        w[j] = writeback(j)
    for j in range(max(0, _NCHUNK - _NB), _NCHUNK):
        w[j].wait()


def kernel(x, embed_matrix):
    idx = x.astype(jnp.int32).T.reshape(-1)
    out = _gather_kernel(idx, embed_matrix)
    return out.reshape(_F, _B, _D).transpose(1, 0, 2)
